# baseline (speedup 1.0000x reference)
import jax
import jax.numpy as jnp
from jax.experimental import pallas as pl
from jax.experimental.pallas import tpu as pltpu
from jax.sharding import Mesh, PartitionSpec as P


_TM = 2304


def _dist_kernel(f_ref, c_ref, o_ref, cbf_ref, csq_ref):
    @pl.when(pl.program_id(0) == 0)
    def _():
        c = c_ref[...]
        cbf_ref[...] = c.astype(jnp.bfloat16)
        csq_ref[...] = jnp.sum(c * c, axis=1, keepdims=True).reshape(1, -1)

    f = f_ref[...]
    f_sq = jnp.sum(f * f, axis=1, keepdims=True)
    fneg = (-2.0 * f).astype(jnp.bfloat16)
    cross = jax.lax.dot_general(
        fneg, cbf_ref[...],
        dimension_numbers=(((1,), (1,)), ((), ())),
        preferred_element_type=jnp.float32)
    o_ref[...] = cross + f_sq + csq_ref[...]


def _dist_call(x, Ck):
    B, S, D = x.shape
    K = Ck.shape[1]
    M = B * S
    f = x.reshape(M, D)
    c = Ck.reshape(K, D)
    tm = _TM if M % _TM == 0 else M
    out = pl.pallas_call(
        _dist_kernel,
        grid=(M // tm,),
        in_specs=[
            pl.BlockSpec((tm, D), lambda i: (i, 0)),
            pl.BlockSpec((K, D), lambda i: (0, 0)),
        ],
        out_specs=pl.BlockSpec((tm, K), lambda i: (i, 0)),
        out_shape=jax.ShapeDtypeStruct((M, K), jnp.float32),
        scratch_shapes=[
            pltpu.VMEM((K, D), jnp.bfloat16),
            pltpu.VMEM((1, K), jnp.float32),
        ],
    )(f, c)
    return out.reshape(B, S, K)


def kernel(x, Ck):
    devices = jax.devices()
    if len(devices) >= 2 and x.shape[0] % 2 == 0:
        mesh = Mesh(devices[:2], ("d",))
        fn = jax.shard_map(
            _dist_call,
            mesh=mesh,
            in_specs=(P("d", None, None), P(None, None, None)),
            out_specs=P("d", None, None),
            check_vma=False,
        )
        return fn(x, Ck)
    return _dist_call(x, Ck)


# speedup vs baseline: 21.2799x; 21.2799x over previous
import jax
import jax.numpy as jnp
from jax.experimental import pallas as pl
from jax.experimental.pallas import tpu as pltpu


_TM = 2304


def _hi_lo(v):
    hi = v.astype(jnp.bfloat16)
    lo = (v - hi.astype(jnp.float32)).astype(jnp.bfloat16)
    return hi, lo


def _dist_kernel(f_ref, c_ref, o_ref, caug_ref):
    @pl.when(pl.program_id(0) == 0)
    def _():
        c = c_ref[...]
        csq = jnp.sum(c * c, axis=1, keepdims=True)
        chi, clo = _hi_lo(csq)
        one = jnp.ones(csq.shape, jnp.bfloat16)
        caug_ref[...] = jnp.concatenate(
            [c.astype(jnp.bfloat16), one, one, chi, clo], axis=1)

    f = f_ref[...]
    f_sq = jnp.sum(f * f, axis=1, keepdims=True)
    fhi, flo = _hi_lo(f_sq)
    fneg = (-2.0 * f).astype(jnp.bfloat16)
    one = jnp.ones(f_sq.shape, jnp.bfloat16)
    faug = jnp.concatenate([fneg, fhi, flo, one, one], axis=1)
    o_ref[...] = jax.lax.dot_general(
        faug, caug_ref[...],
        dimension_numbers=(((1,), (1,)), ((), ())),
        preferred_element_type=jnp.float32)


def kernel(x, Ck):
    B, S, D = x.shape
    K = Ck.shape[1]
    M = B * S
    f = x.reshape(M, D)
    c = Ck.reshape(K, D)
    tm = _TM if M % _TM == 0 else M
    out = pl.pallas_call(
        _dist_kernel,
        grid=(M // tm,),
        in_specs=[
            pl.BlockSpec((tm, D), lambda i: (i, 0)),
            pl.BlockSpec((K, D), lambda i: (0, 0)),
        ],
        out_specs=pl.BlockSpec((tm, K), lambda i: (i, 0)),
        out_shape=jax.ShapeDtypeStruct((M, K), jnp.float32),
        scratch_shapes=[
            pltpu.VMEM((K, D + 4), jnp.bfloat16),
        ],
    )(f, c)
    return out.reshape(B, S, K)


# speedup vs baseline: 23.4452x; 1.1017x over previous
import jax
import jax.numpy as jnp
from jax.experimental import pallas as pl
from jax.experimental.pallas import tpu as pltpu


_TM = 2304


def _dist_kernel(f_ref, c_ref, o_ref, cbf_ref, csq_ref):
    @pl.when(pl.program_id(0) == 0)
    def _():
        c = c_ref[...]
        cbf_ref[...] = c.astype(jnp.bfloat16)
        csq_ref[...] = jnp.sum(c * c, axis=1, keepdims=True).reshape(1, -1)

    f = f_ref[...]
    f_sq = jnp.sum(f * f, axis=1, keepdims=True)
    fneg = (-2.0 * f).astype(jnp.bfloat16)
    cross = jax.lax.dot_general(
        fneg, cbf_ref[...],
        dimension_numbers=(((1,), (1,)), ((), ())),
        preferred_element_type=jnp.float32)
    o_ref[...] = cross + f_sq + csq_ref[...]


def kernel(x, Ck):
    B, S, D = x.shape
    K = Ck.shape[1]
    M = B * S
    f = x.reshape(M, D)
    c = Ck.reshape(K, D)
    tm = _TM if M % _TM == 0 else M
    out = pl.pallas_call(
        _dist_kernel,
        grid=(M // tm,),
        in_specs=[
            pl.BlockSpec((tm, D), lambda i: (i, 0)),
            pl.BlockSpec((K, D), lambda i: (0, 0)),
        ],
        out_specs=pl.BlockSpec((tm, K), lambda i: (i, 0)),
        out_shape=jax.ShapeDtypeStruct((M, K), jnp.float32),
        scratch_shapes=[
            pltpu.VMEM((K, D), jnp.bfloat16),
            pltpu.VMEM((1, K), jnp.float32),
        ],
    )(f, c)
    return out.reshape(B, S, K)
